# baseline (speedup 1.0000x reference)
import functools

import jax
import jax.numpy as jnp
from jax import lax
from jax.experimental import pallas as pl
from jax.experimental.pallas import tpu as pltpu
from jax.experimental.pallas import tpu_sc as plsc

VOCAB = 1000
VPAD = 1024
B, T = 1024, 50
TPAD = 56
NW = 32
BPW = B // NW


def _make_gather():
  mesh = plsc.VectorSubcoreMesh(core_axis_name="c", subcore_axis_name="s")

  @functools.partial(
      pl.kernel,
      out_type=jax.ShapeDtypeStruct((B, TPAD, VPAD), jnp.float32),
      mesh=mesh,
      scratch_types=[
          pltpu.VMEM((BPW, TPAD), jnp.int32),
          pltpu.VMEM((2, TPAD, VPAD), jnp.float32),
          pltpu.SemaphoreType.DMA,
          pltpu.SemaphoreType.DMA,
      ],
      compiler_params=pltpu.CompilerParams(use_tc_tiling_on_sc=True),
  )
  def k(idx_hbm, table_hbm, out_hbm, idx_v, rows_v, sem0, sem1):
    wid = lax.axis_index("s") * 2 + lax.axis_index("c")
    base = wid * BPW
    sems = (sem0, sem1)

    pltpu.sync_copy(idx_hbm.at[pl.ds(base, BPW)], idx_v)

    def gather(c, b):
      return pltpu.make_async_copy(
          table_hbm.at[idx_v.at[c]], rows_v.at[b], sems[b])

    gather(0, 0).start()

    def outer(i2, _):
      for b in range(2):
        c = i2 * 2 + b
        nxt = c + 1

        @pl.when(nxt < BPW)
        def _():
          gather(nxt, 1 - b).start()

        gather(c, b).wait()
        pltpu.sync_copy(rows_v.at[b], out_hbm.at[base + c])
      return _

    lax.fori_loop(0, BPW // 2, outer, None)

  return k


_gather = _make_gather()


def kernel(idx, table):
  table_p = jnp.pad(table, ((0, 0), (0, VPAD - VOCAB)))
  idx_p = jnp.pad(idx.astype(jnp.int32), ((0, 0), (0, TPAD - T)))
  out_p = _gather(idx_p, table_p)
  return out_p[:, :T, :VOCAB]


# speedup vs baseline: 2.2604x; 2.2604x over previous
import functools

import jax
import jax.numpy as jnp
from jax import lax
from jax.experimental import pallas as pl
from jax.experimental.pallas import tpu as pltpu
from jax.experimental.pallas import tpu_sc as plsc

VOCAB = 1000
VPAD = 1024
B, T = 1024, 50
TPAD = 56
NW = 32
BPW = B // NW


def _make_gather():
  mesh = plsc.VectorSubcoreMesh(core_axis_name="c", subcore_axis_name="s")

  @functools.partial(
      pl.kernel,
      out_type=jax.ShapeDtypeStruct((B, TPAD, VPAD), jnp.float32),
      mesh=mesh,
      scratch_types=[
          pltpu.VMEM((BPW, TPAD), jnp.int32),
          pltpu.VMEM((2, TPAD, VPAD), jnp.float32),
          pltpu.SemaphoreType.DMA,
          pltpu.SemaphoreType.DMA,
      ],
      compiler_params=pltpu.CompilerParams(use_tc_tiling_on_sc=True),
  )
  def k(idx_hbm, table_hbm, out_hbm, idx_v, rows_v, sem0, sem1):
    wid = lax.axis_index("s") * 2 + lax.axis_index("c")
    base = wid * BPW
    sems = (sem0, sem1)

    pltpu.sync_copy(idx_hbm.at[pl.ds(base, BPW)], idx_v)

    def gather(c, b):
      return pltpu.make_async_copy(
          table_hbm.at[idx_v.at[c]], rows_v.at[b], sems[b])

    gather(0, 0).start()

    def outer(i2, _):
      for b in range(2):
        c = i2 * 2 + b
        nxt = c + 1

        @pl.when(nxt < BPW)
        def _():
          gather(nxt, 1 - b).start()

        gather(c, b).wait()
        pltpu.sync_copy(rows_v.at[b], out_hbm.at[base + c])
      return _

    lax.fori_loop(0, BPW // 2, outer, None)

  return k


_gather = _make_gather()


def kernel(idx, table):
  table_p = jnp.pad(table, ((0, 0), (0, VPAD - VOCAB)))
  idx_p = jnp.concatenate(
      [idx.astype(jnp.int32), idx.astype(jnp.int32)[:, :TPAD - T]], axis=1)
  out_p = _gather(idx_p, table_p)
  return out_p[:, :T, :VOCAB]
